# baseline (speedup 1.0000x reference)
import functools

import jax
import jax.numpy as jnp
from jax import lax
from jax.experimental import pallas as pl
from jax.experimental.pallas import tpu as pltpu
from jax.experimental.pallas import tpu_sc as plsc

BATCH = 4
CHANNELS = 256
HH = 56
WW = 56
SPAD = 64
RATS = ((8, 8), (12, 12), (16, 16))
NSEL = (2, 3, 2)
OUTS = (49, 45, 41)
HGWG = (2401, 2025, 1681)
GOFF = (0, 2401, 4426)
GOFFP = (0, 2408, 4440)
WSROW = 6144
NCHS = (151, 127, 106)
MAGIC = (1338, 1457, 1599)
HCH = (4, 3, 3)
NEG_INF = float("-inf")


def _tc_body(x_ref, smap_ref):
    smap = jnp.sum(x_ref[0], axis=0)
    pad = jnp.zeros((HH, SPAD - WW), jnp.float32)
    smap_ref[0] = jnp.concatenate([smap, pad], axis=1)


def _tc_stage(x):
    return pl.pallas_call(
        _tc_body,
        grid=(BATCH,),
        in_specs=[pl.BlockSpec((1, CHANNELS, HH, WW), lambda b: (b, 0, 0, 0))],
        out_specs=pl.BlockSpec((1, HH, SPAD), lambda b: (b, 0, 0)),
        out_shape=jax.ShapeDtypeStruct((BATCH, HH, SPAD), jnp.float32),
        compiler_params=pltpu.CompilerParams(
            dimension_semantics=("parallel",)),
    )(x)


@functools.partial(
    pl.kernel,
    mesh=plsc.VectorSubcoreMesh(core_axis_name="c", subcore_axis_name="s"),
    out_type=[
        jax.ShapeDtypeStruct((BATCH * WSROW,), jnp.float32),
        jax.ShapeDtypeStruct((96,), jnp.int32),
        jax.ShapeDtypeStruct((96,), jnp.float32),
    ],
    scratch_types=[
        pltpu.VMEM((HH * SPAD,), jnp.float32),
        pltpu.VMEM((3200,), jnp.float32),
        pltpu.VMEM((2432,), jnp.float32),
        pltpu.VMEM((16,), jnp.int32),
        pltpu.VMEM((16,), jnp.float32),
    ],
    compiler_params=pltpu.CompilerParams(needs_layout_passes=False),
)
def _sc_stage(smap_hbm, ws_hbm, idx_hbm, sc_hbm, smap_v, vert_v, pool_v,
              idxb, scb):
    cid = lax.axis_index("c")
    sid = lax.axis_index("s")
    wid = sid * 2 + cid
    lane = lax.iota(jnp.int32, 16)
    for g in range(3):
        h, w = RATS[g]
        hg = OUTS[g]

        @pl.when((wid >= 4 * g) & (wid < 4 * g + 4))
        def _(g=g, h=h, w=w, hg=hg):
            nsel = NSEL[g]
            nwin = HGWG[g]
            nch = NCHS[g]
            magic = MAGIC[g]
            inv_area = 1.0 / float(h * w)
            b = wid - 4 * g
            pltpu.sync_copy(smap_hbm.at[pl.ds(b * HH * SPAD, HH * SPAD)],
                            smap_v)

            def vfn(r, _):
                for c0 in range(0, SPAD, 16):
                    acc = smap_v[pl.ds(r * SPAD + c0, 16)]
                    for di in range(1, h):
                        acc = acc + smap_v[pl.ds((r + di) * SPAD + c0, 16)]
                    vert_v[pl.ds(r * SPAD + c0, 16)] = acc
                return 0

            lax.fori_loop(0, hg, vfn, 0)

            def hfn(r, _):
                for j0 in range(0, HCH[g] * 16, 16):
                    acc = vert_v[pl.ds(r * SPAD + j0, 16)]
                    for dj in range(1, w):
                        acc = acc + vert_v[pl.ds(r * SPAD + j0 + dj, 16)]
                    pool_v[pl.ds(r * hg + j0, 16)] = acc * inv_area
                return 0

            lax.fori_loop(0, hg, hfn, 0)
            pool_v[pl.ds(nwin, 16)] = jnp.full((16,), NEG_INF, jnp.float32)

            pltpu.sync_copy(pool_v.at[pl.ds(0, nwin)],
                            ws_hbm.at[pl.ds(b * WSROW + GOFFP[g], nwin)])

            sels, scs = [], []
            for t in range(nsel):
                def scanfn(i, carry):
                    vmax, lastc = carry
                    v = pool_v[pl.ds(i * 16, 16)]
                    upd = v >= vmax
                    return (jnp.maximum(vmax, v),
                            jnp.where(upd, i, lastc))

                vmax, lastc = lax.fori_loop(
                    0, nch, scanfn,
                    (jnp.full((16,), NEG_INF, jnp.float32),
                     jnp.zeros((16,), jnp.int32)))
                mx = jnp.max(vmax)
                flat = lastc * 16 + lane
                sel = jnp.max(jnp.where(vmax == mx, flat, jnp.int32(-1)))
                si = (sel * magic) >> 16
                sj = sel - si * hg
                sels.append(GOFF[g] + sel)
                scs.append(mx)

                if t < nsel - 1:
                    k0 = (jnp.maximum(si - (h - 1), 0) * hg) >> 4
                    k1 = (jnp.minimum(si + h, hg) * hg + 15) >> 4

                    def killfn(i, _):
                        q = i * 16 + lane
                        rr = (q * magic) >> 16
                        cc = q - rr * hg
                        dl1 = jnp.maximum(h - jnp.abs(rr - si), 0)
                        dl2 = jnp.maximum(w - jnp.abs(cc - sj), 0)
                        kill = (5 * dl1 * dl2 > 2 * h * w) | (q == sel)
                        v = pool_v[pl.ds(i * 16, 16)]
                        pool_v[pl.ds(i * 16, 16)] = jnp.where(
                            kill, NEG_INF, v)
                        return 0

                    lax.fori_loop(k0, k1, killfn, 0)

            idxvec = jnp.zeros((16,), jnp.int32)
            scvec = jnp.zeros((16,), jnp.float32)
            for t in range(nsel):
                idxvec = jnp.where(lane == t, sels[t], idxvec)
                scvec = jnp.where(lane == t, scs[t], scvec)
            idxb[...] = idxvec
            scb[...] = scvec
            t_out = (b * 3 + g) * 8
            pltpu.sync_copy(idxb.at[pl.ds(0, 8)], idx_hbm.at[pl.ds(t_out, 8)])
            pltpu.sync_copy(scb.at[pl.ds(0, 8)], sc_hbm.at[pl.ds(t_out, 8)])


def kernel(x, coordinates_cat):
    smap = _tc_stage(x)
    wsp, idx, sc = _sc_stage(smap.reshape(-1))
    wsp = wsp.reshape(BATCH, WSROW)
    window_scores = jnp.concatenate(
        [wsp[:, GOFFP[g]:GOFFP[g] + HGWG[g]] for g in range(3)], axis=1)
    idx = idx.reshape(BATCH, 3, 8)
    sc = sc.reshape(BATCH, 3, 8)
    proposal_idx = jnp.concatenate(
        [idx[:, 0, :2], idx[:, 1, :3], idx[:, 2, :2]], axis=1)
    proposal_sc = jnp.concatenate(
        [sc[:, 0, :2], sc[:, 1, :3], sc[:, 2, :2]], axis=1)
    return (proposal_idx, proposal_sc, window_scores)


# speedup vs baseline: 1.1215x; 1.1215x over previous
import functools

import jax
import jax.numpy as jnp
from jax import lax
from jax.experimental import pallas as pl
from jax.experimental.pallas import tpu as pltpu
from jax.experimental.pallas import tpu_sc as plsc

BATCH = 4
CHANNELS = 256
HH = 56
WW = 56
RATS = ((8, 8), (12, 12), (16, 16))
NSEL = (2, 3, 2)
OUTS = (49, 45, 41)
GOFF = (0, 2401, 4426)
ROWOFF = (0, 49, 94)
NROWS = 135
WPAD = 64
NEG_INF = float("-inf")


def _tc_body(x_ref, maps_ref):
    smap = jnp.sum(x_ref[0], axis=0)
    for (h, w), out_w, r0 in zip(RATS, OUTS, ROWOFF):
        acc = smap[:, 0:out_w]
        for dj in range(1, w):
            acc = acc + smap[:, dj:dj + out_w]
        accv = acc[0:out_w, :]
        for di in range(1, h):
            accv = accv + acc[di:di + out_w, :]
        pooled = accv / jnp.float32(h * w)
        pad = jnp.full((out_w, WPAD - out_w), NEG_INF, jnp.float32)
        maps_ref[0, r0:r0 + out_w] = jnp.concatenate([pooled, pad], axis=1)


def _tc_stage(x):
    return pl.pallas_call(
        _tc_body,
        grid=(BATCH,),
        in_specs=[pl.BlockSpec((1, CHANNELS, HH, WW), lambda b: (b, 0, 0, 0))],
        out_specs=pl.BlockSpec((1, NROWS, WPAD), lambda b: (b, 0, 0)),
        out_shape=jax.ShapeDtypeStruct((BATCH, NROWS, WPAD), jnp.float32),
        compiler_params=pltpu.CompilerParams(
            dimension_semantics=("parallel",)),
    )(x)


@functools.partial(
    pl.kernel,
    mesh=plsc.VectorSubcoreMesh(core_axis_name="c", subcore_axis_name="s"),
    out_type=[
        jax.ShapeDtypeStruct((96,), jnp.int32),
        jax.ShapeDtypeStruct((96,), jnp.float32),
    ],
    scratch_types=[
        pltpu.VMEM((OUTS[0] * WPAD,), jnp.float32),
        pltpu.VMEM((16,), jnp.int32),
        pltpu.VMEM((16,), jnp.float32),
    ],
    compiler_params=pltpu.CompilerParams(needs_layout_passes=False),
)
def _sc_nms(maps_hbm, idx_hbm, sc_hbm, map_v, idxb, scb):
    cid = lax.axis_index("c")
    sid = lax.axis_index("s")
    wid = sid * 2 + cid
    lane = lax.iota(jnp.int32, 16)
    for g in range(3):
        h, w = RATS[g]
        hg = OUTS[g]
        nsel = NSEL[g]
        nch = hg * (WPAD // 16)

        @pl.when((wid >= 4 * g) & (wid < 4 * g + 4))
        def _(g=g, h=h, w=w, hg=hg, nsel=nsel, nch=nch):
            b = wid - 4 * g
            n = hg * WPAD
            src0 = b * NROWS * WPAD + ROWOFF[g] * WPAD
            pltpu.sync_copy(maps_hbm.at[pl.ds(src0, n)], map_v.at[pl.ds(0, n)])
            sels, scs = [], []

            for t in range(nsel):
                def scanfn(i, carry):
                    vmax, lastc = carry
                    v = map_v[pl.ds(i * 16, 16)]
                    upd = v >= vmax
                    return (jnp.maximum(vmax, v),
                            jnp.where(upd, i, lastc))

                vmax, lastc = lax.fori_loop(
                    0, nch, scanfn,
                    (jnp.full((16,), NEG_INF, jnp.float32),
                     jnp.zeros((16,), jnp.int32)))
                mx = jnp.max(vmax)
                flat = lastc * 16 + lane
                selflat = jnp.max(jnp.where(vmax == mx, flat, jnp.int32(-1)))
                si = selflat >> 6
                sj = selflat & 63
                sels.append(GOFF[g] + si * hg + sj)
                scs.append(mx)

                if t < nsel - 1:
                    k0 = jnp.maximum(si - (h - 1), 0) * (WPAD // 16)
                    k1 = jnp.minimum(si + h, hg) * (WPAD // 16)

                    def killfn(i, _):
                        rr = i >> 2
                        cc = (i & 3) * 16 + lane
                        dl1 = jnp.maximum(h - jnp.abs(rr - si), 0)
                        dl2 = jnp.maximum(w - jnp.abs(cc - sj), 0)
                        kill = ((5 * dl1 * dl2 > 2 * h * w)
                                | ((rr == si) & (cc == sj)))
                        v = map_v[pl.ds(i * 16, 16)]
                        map_v[pl.ds(i * 16, 16)] = jnp.where(kill, NEG_INF, v)
                        return 0

                    lax.fori_loop(k0, k1, killfn, 0)

            idxvec = jnp.zeros((16,), jnp.int32)
            scvec = jnp.zeros((16,), jnp.float32)
            for t in range(nsel):
                idxvec = jnp.where(lane == t, sels[t], idxvec)
                scvec = jnp.where(lane == t, scs[t], scvec)
            idxb[...] = idxvec
            scb[...] = scvec
            t_out = (b * 3 + g) * 8
            pltpu.sync_copy(idxb.at[pl.ds(0, 8)], idx_hbm.at[pl.ds(t_out, 8)])
            pltpu.sync_copy(scb.at[pl.ds(0, 8)], sc_hbm.at[pl.ds(t_out, 8)])


def kernel(x, coordinates_cat):
    maps = _tc_stage(x)
    idx, sc = _sc_nms(maps.reshape(-1))
    window_scores = jnp.concatenate(
        [maps[:, ROWOFF[g]:ROWOFF[g] + OUTS[g], :OUTS[g]].reshape(BATCH, -1)
         for g in range(3)], axis=1)
    idx = idx.reshape(BATCH, 3, 8)
    sc = sc.reshape(BATCH, 3, 8)
    proposal_idx = jnp.concatenate(
        [idx[:, 0, :2], idx[:, 1, :3], idx[:, 2, :2]], axis=1)
    proposal_sc = jnp.concatenate(
        [sc[:, 0, :2], sc[:, 1, :3], sc[:, 2, :2]], axis=1)
    return (proposal_idx, proposal_sc, window_scores)
